# baseline (speedup 1.0000x reference)
import jax
import jax.numpy as jnp
from jax.experimental import pallas as pl
from jax.experimental.pallas import tpu as pltpu

_MAXC = 64
_HID = 256
_T = 4
_ITERS = 2
_NSEM = 57
_NBLK = 8
_CPB = _MAXC // _NBLK


def _body(p1_ref, p2_ref, p3_ref, wp_ref, bp_ref,
          wex_ref, bex_ref, wsem_ref, bsem_ref, wel_ref, bel_ref,
          wee_ref, bee_ref, wne_ref, bne_ref, wc_ref, bc_ref,
          wc2_ref, bc2_ref,
          out_ref, sem_ref, ex_ref, el_out_ref, cf_scr):
    f32 = jnp.float32
    s = pl.program_id(0)
    pf = jnp.concatenate([p1_ref[...], p2_ref[...], p3_ref[...]], axis=1)

    for c in range(_CPB):
        row = jnp.maximum(
            jnp.dot(pf, wp_ref[:, c, :], preferred_element_type=f32)
            + bp_ref[c:c + 1, :], 0.0)
        cf_scr[pl.ds(s * _CPB + c, 1), :] = row

    @pl.when(s == _NBLK - 1)
    def _main():
        cf0 = cf_scr[...]

        exl = jnp.dot(cf0, wex_ref[...], preferred_element_type=f32) \
            + bex_ref[...]
        ex_ref[0] = exl
        exists0 = exl > 0.0

        P = jnp.dot(cf0, wel_ref[:_HID, :], preferred_element_type=f32)
        Q = jnp.dot(cf0, wel_ref[_HID:, :], preferred_element_type=f32)
        el = jnp.maximum(P[:, None, :] + Q[None, :, :]
                         + bel_ref[...][None, :, :], 0.0)
        el2 = el.reshape(_MAXC * _MAXC, _HID)

        L2 = jax.lax.dot_general(
            el2, wee_ref[...], (((1,), (1,)), ((), ())),
            preferred_element_type=f32)
        L3 = L2.reshape(_MAXC, _MAXC, _T) + bee_ref[...]
        el_out_ref[0] = L3
        mask3 = ((L3 > 0.0)
                 & exists0[:, :, None]
                 & exists0.reshape(1, _MAXC, 1)).astype(f32)
        num_edges = jnp.sum(mask3)

        cf = cf0
        cf_list = [cf0]
        for it in range(_ITERS):
            A = jnp.dot(cf, wne_ref[it, :_HID, :],
                        preferred_element_type=f32)
            Bv = jnp.dot(cf, wne_ref[it, _HID:2 * _HID, :],
                         preferred_element_type=f32)
            C = jnp.dot(el2, wne_ref[it, 2 * _HID:3 * _HID, :],
                        preferred_element_type=f32)
            C = C.reshape(_MAXC, _MAXC, _HID)
            wt4 = wne_ref[it, 3 * _HID:3 * _HID + _T, :]
            bne = bne_ref[it:it + 1, :][None]
            base = A[:, None, :] + Bv[None, :, :] + C + bne
            acc = jnp.zeros((_MAXC, _HID), f32)
            for t in range(_T):
                lt = L3[:, :, t:t + 1]
                wrow = wt4[t:t + 1, :][None]
                r = jnp.maximum(base + lt * wrow, 0.0)
                acc = acc + jnp.sum(r * mask3[:, :, t:t + 1], axis=1)
            cf = jnp.where(num_edges > 0.0, acc, cf)
            cf_list.append(cf)

        h = jnp.maximum(
            jnp.dot(cf_list[0], wc_ref[:_HID, :], preferred_element_type=f32)
            + jnp.dot(cf_list[1], wc_ref[_HID:2 * _HID, :],
                      preferred_element_type=f32)
            + jnp.dot(cf_list[2], wc_ref[2 * _HID:, :],
                      preferred_element_type=f32)
            + bc_ref[...], 0.0)
        sem_ref[0] = jnp.dot(h, wsem_ref[...],
                             preferred_element_type=f32) + bsem_ref[...]
        out_ref[0] = jnp.maximum(
            jnp.dot(h, wc2_ref[...], preferred_element_type=f32)
            + bc2_ref[...], 0.0)


def kernel(parent_feature, gt_children_code, gt_num_code, W_parent, b_parent,
           W_exists, b_exists, W_sem, b_sem, W_edge_latent, b_edge_latent,
           W_edge_exists, b_edge_exists, W_node_edge, b_node_edge,
           W_child, b_child, W_child2, b_child2):
    feat = parent_feature.shape[1]
    n1 = parent_feature.shape[1]
    n2 = gt_children_code.shape[1]
    n3 = gt_num_code.shape[1]
    pin = n1 + n2 + n3

    wp3 = W_parent.reshape(pin, _MAXC, _HID)
    bp2 = b_parent.reshape(_MAXC, _HID)

    out_shapes = (
        jax.ShapeDtypeStruct((1, _MAXC, feat), jnp.float32),
        jax.ShapeDtypeStruct((1, _MAXC, _NSEM), jnp.float32),
        jax.ShapeDtypeStruct((1, _MAXC, 1), jnp.float32),
        jax.ShapeDtypeStruct((1, _MAXC, _MAXC, _T), jnp.float32),
    )
    full = lambda a: pl.BlockSpec(a.shape, lambda s: tuple(0 for _ in a.shape))

    args = (parent_feature, gt_children_code, gt_num_code)
    warr = (W_exists, b_exists.reshape(1, 1), W_sem, b_sem.reshape(1, _NSEM),
            W_edge_latent, b_edge_latent.reshape(1, _HID),
            W_edge_exists, b_edge_exists.reshape(1, 1, _T),
            W_node_edge, b_node_edge,
            W_child, b_child.reshape(1, _HID),
            W_child2, b_child2.reshape(1, feat))

    in_specs = [full(a) for a in args]
    in_specs += [
        pl.BlockSpec((pin, _CPB, _HID), lambda s: (0, s, 0)),
        pl.BlockSpec((_CPB, _HID), lambda s: (s, 0)),
    ]
    in_specs += [full(a) for a in warr]

    out_specs = tuple(
        pl.BlockSpec(sh.shape, lambda s, _n=len(sh.shape): tuple(0 for _ in range(_n)))
        for sh in out_shapes)

    outs = pl.pallas_call(
        _body,
        grid=(_NBLK,),
        in_specs=in_specs,
        out_specs=list(out_specs),
        out_shape=list(out_shapes),
        scratch_shapes=[pltpu.VMEM((_MAXC, _HID), jnp.float32)],
    )(*args, wp3, bp2, *warr)
    return tuple(outs)
